# baseline (speedup 1.0000x reference)
import functools

import jax
import jax.numpy as jnp
from jax import lax
from jax.experimental import pallas as pl
from jax.experimental.pallas import tpu as pltpu
from jax.experimental.pallas import tpu_sc as plsc

EMB = 32
NUM_CORES = 2
NUM_SUBCORES = 16
LANES = 16
NUM_WORKERS = NUM_CORES * NUM_SUBCORES


def _mf_sc(user_id, game_id, user_table, game_table):
    batch = user_id.shape[0]
    bpw = batch // NUM_WORKERS
    mesh = plsc.VectorSubcoreMesh(core_axis_name="c", subcore_axis_name="s")

    @functools.partial(
        pl.kernel,
        mesh=mesh,
        out_type=jax.ShapeDtypeStruct((batch,), jnp.float32),
        compiler_params=pltpu.CompilerParams(
            needs_layout_passes=False, use_tc_tiling_on_sc=False),
        scratch_types=[
            pltpu.VMEM((bpw,), jnp.int32),
            pltpu.VMEM((bpw,), jnp.int32),
            pltpu.VMEM((bpw, EMB), jnp.float32),
            pltpu.VMEM((bpw, EMB), jnp.float32),
            pltpu.VMEM((bpw,), jnp.float32),
            pltpu.SemaphoreType.DMA,
            pltpu.SemaphoreType.DMA,
        ],
    )
    def mf_kernel(uid_hbm, gid_hbm, ut_hbm, gt_hbm, out_hbm,
                  uid_v, gid_v, u_v, g_v, o_v, sem_u, sem_g):
        wid = lax.axis_index("s") * NUM_CORES + lax.axis_index("c")
        base = wid * bpw
        pltpu.sync_copy(uid_hbm.at[pl.ds(base, bpw)], uid_v)
        pltpu.sync_copy(gid_hbm.at[pl.ds(base, bpw)], gid_v)
        cp_u = pltpu.async_copy(ut_hbm.at[uid_v], u_v, sem_u)
        cp_g = pltpu.async_copy(gt_hbm.at[gid_v], g_v, sem_g)
        cp_u.wait()
        cp_g.wait()

        lanes = lax.iota(jnp.int32, LANES)

        @pl.loop(0, bpw, step=LANES)
        def _(r0):
            rows = r0 + lanes
            acc = jnp.zeros((LANES,), jnp.float32)
            for j in range(EMB):
                cols = jnp.full((LANES,), j, jnp.int32)
                u_col = plsc.load_gather(u_v, [rows, cols])
                g_col = plsc.load_gather(g_v, [rows, cols])
                acc = acc + u_col * g_col
            o_v[pl.ds(r0, LANES)] = 10.0 / (1.0 + jnp.exp(-acc))

        pltpu.sync_copy(o_v, out_hbm.at[pl.ds(base, bpw)])

    return mf_kernel(user_id, game_id, user_table, game_table)


def kernel(user_id, game_id, user_table, game_table):
    user_id = user_id.astype(jnp.int32)
    game_id = game_id.astype(jnp.int32)
    return _mf_sc(user_id, game_id, user_table, game_table)
